# baseline (speedup 1.0000x reference)
import functools

import jax
import jax.numpy as jnp
from jax import lax
from jax.experimental import pallas as pl
from jax.experimental.pallas import tpu as pltpu
from jax.experimental.pallas import tpu_sc as plsc

NUM_CORES = 2
NUM_SUBCORES = 16
NUM_WORKERS = NUM_CORES * NUM_SUBCORES
LANES = 16


@functools.partial(jax.jit, static_argnums=(2, 3))
def _bow_encode(input_flat, emb_weight, B, S):
    V, E = emb_weight.shape
    b_per_w = B // NUM_WORKERS
    n_chunks = 5
    chunk = S // n_chunks
    n_col = E // LANES

    mesh = plsc.VectorSubcoreMesh(
        core_axis_name="c", subcore_axis_name="s",
        num_cores=NUM_CORES, num_subcores=NUM_SUBCORES,
    )

    @functools.partial(
        pl.kernel,
        out_type=jax.ShapeDtypeStruct((B, E), jnp.float32),
        mesh=mesh,
        scratch_types=[
            pltpu.VMEM((b_per_w * S,), jnp.int32),
            pltpu.VMEM((S, E), jnp.float32),
            pltpu.VMEM((S, E), jnp.float32),
            pltpu.VMEM((b_per_w, E), jnp.float32),
            pltpu.SemaphoreType.DMA,
            pltpu.SemaphoreType.DMA,
        ],
        compiler_params=pltpu.CompilerParams(use_tc_tiling_on_sc=False),
    )
    def bow_kernel(idx_hbm, table_hbm, out_hbm, idx_v, buf0, buf1, out_v,
                   sem0, sem1):
        wid = lax.axis_index("s") * NUM_CORES + lax.axis_index("c")
        base = wid * b_per_w
        pltpu.sync_copy(idx_hbm.at[pl.ds(base * S, b_per_w * S)], idx_v)

        bufs = (buf0, buf1)
        sems = (sem0, sem1)

        def fire(b, p):
            for j in range(n_chunks):
                off = pl.multiple_of(b * S + j * chunk, 8)
                pltpu.async_copy(
                    table_hbm.at[idx_v.at[pl.ds(off, chunk)]],
                    bufs[p].at[pl.ds(j * chunk, chunk), :],
                    sems[p],
                )

        def drain(p):
            pltpu.make_async_copy(
                table_hbm.at[pl.ds(0, S), :], bufs[p], sems[p]).wait()

        def reduce(p, b):
            buf = bufs[p]

            def body(s, accs):
                return tuple(
                    jnp.maximum(a, buf[s, pl.ds(j * LANES, LANES)])
                    for j, a in enumerate(accs)
                )

            accs = tuple(buf[0, pl.ds(j * LANES, LANES)] for j in range(n_col))
            accs = lax.fori_loop(1, S, body, accs, unroll=4)
            for j in range(n_col):
                out_v[b, pl.ds(j * LANES, LANES)] = accs[j]

        fire(0, 0)

        def pair_body(i):
            b0 = 2 * i
            fire(b0 + 1, 1)
            drain(0)
            reduce(0, b0)
            fire(b0 + 2, 0)
            drain(1)
            reduce(1, b0 + 1)

        pl.loop(0, b_per_w // 2 - 1)(pair_body)

        b0 = b_per_w - 2
        fire(b0 + 1, 1)
        drain(0)
        reduce(0, b0)
        drain(1)
        reduce(1, b0 + 1)

        pltpu.sync_copy(out_v, out_hbm.at[pl.ds(base, b_per_w), :])

    return bow_kernel(input_flat, emb_weight)


def kernel(input, emb_weight):
    B, S = input.shape
    flat = input.astype(jnp.int32).reshape(B * S)
    return _bow_encode(flat, emb_weight, B, S)
